# baseline (speedup 1.0000x reference)
import jax
import jax.numpy as jnp
from jax import lax
from jax.experimental import pallas as pl
from jax.experimental.pallas import tpu as pltpu

B, L, D_IN, D_H, D_OUT, TOPK = 4, 8192, 128, 64, 22, 8
D_PAD = 24
NCHUNK = 8
WINDOW = 3
CT = (B * L) // NCHUNK
CPB = L // CT
SUB = 1024


def _rowmax(g):
    g = jnp.maximum(g, pltpu.roll(g, 4, axis=0))
    g = jnp.maximum(g, pltpu.roll(g, 2, axis=0))
    return jnp.maximum(g, pltpu.roll(g, 1, axis=0))


def _mlp_topk_kernel(x_hbm, w1_ref, b1_ref, w2_ref, b2_ref, o_hbm,
                     x_buf, y_buf, w2p_ref, b2p_ref, in_sems, out_sems):
    w1 = w1_ref[...]
    b1 = b1_ref[...].reshape(1, D_H)
    w2p_ref[...] = jnp.zeros((D_PAD, D_H), jnp.float32)
    w2p_ref[0:D_OUT, :] = w2_ref[...]
    b2p_ref[...] = jnp.zeros((D_PAD, 1), jnp.float32)
    b2p_ref[0:D_OUT, :] = jnp.transpose(b2_ref[...].reshape(1, D_OUT))
    w2 = w2p_ref[...]
    b2 = b2p_ref[...]

    def start_in(i):
        b, j = divmod(i, CPB)
        cp = pltpu.make_async_copy(
            x_hbm.at[b, pl.ds(j * CT, CT), :], x_buf.at[i], in_sems.at[i])
        cp.start()
        return cp

    in_cps = [start_in(i) for i in range(WINDOW)]

    def process(i):
        for s in range(CT // SUB):
            tok = pl.ds(s * SUB, SUB)
            x = x_buf[i, tok, :]
            h = lax.dot_general(x, w1, (((1,), (1,)), ((), ())),
                                preferred_element_type=jnp.float32)
            h = jnp.maximum(h + b1, 0.0)
            y = lax.dot_general(w2, h, (((1,), (1,)), ((), ())),
                                preferred_element_type=jnp.float32)
            y = jnp.maximum(y + b2, 0.0)
            grp = (y[0:8], y[8:16], y[16:24])
            work = grp
            for _ in range(TOPK - 1):
                m = _rowmax(jnp.maximum(jnp.maximum(work[0], work[1]),
                                        work[2]))
                work = tuple(jnp.where(w >= m, -1.0, w) for w in work)
            t8 = _rowmax(jnp.maximum(jnp.maximum(work[0], work[1]), work[2]))
            y_buf[i, 0:8, tok] = jnp.where(grp[0] >= t8, grp[0], 0.0)
            y_buf[i, 8:16, tok] = jnp.where(grp[1] >= t8, grp[1], 0.0)
            y_buf[i, 16:22, tok] = jnp.where(grp[2] >= t8, grp[2], 0.0)[0:6]
        b, j = divmod(i, CPB)
        cp = pltpu.make_async_copy(
            y_buf.at[i],
            o_hbm.at[b, :, pl.ds(j * CT, CT)], out_sems.at[i])
        cp.start()
        return cp

    out_cps = []
    for i in range(NCHUNK):
        in_cps[i].wait()
        if i + WINDOW < NCHUNK:
            in_cps.append(start_in(i + WINDOW))
        out_cps.append(process(i))

    for cp in out_cps:
        cp.wait()


@jax.jit
def kernel(input, W1, b1, W2, b2):
    return pl.pallas_call(
        _mlp_topk_kernel,
        in_specs=[
            pl.BlockSpec(memory_space=pl.ANY),
            pl.BlockSpec(memory_space=pltpu.MemorySpace.VMEM),
            pl.BlockSpec(memory_space=pltpu.MemorySpace.VMEM),
            pl.BlockSpec(memory_space=pltpu.MemorySpace.VMEM),
            pl.BlockSpec(memory_space=pltpu.MemorySpace.VMEM),
        ],
        out_specs=pl.BlockSpec(memory_space=pl.ANY),
        out_shape=jax.ShapeDtypeStruct((B, D_OUT, L), jnp.float32),
        scratch_shapes=[
            pltpu.VMEM((NCHUNK, CT, D_IN), jnp.float32),
            pltpu.VMEM((NCHUNK, D_OUT, CT), jnp.float32),
            pltpu.VMEM((D_PAD, D_H), jnp.float32),
            pltpu.VMEM((D_PAD, 1), jnp.float32),
            pltpu.SemaphoreType.DMA((NCHUNK,)),
            pltpu.SemaphoreType.DMA((NCHUNK,)),
        ],
    )(input, W1, b1, W2, b2)


# speedup vs baseline: 1.0303x; 1.0303x over previous
import jax
import jax.numpy as jnp
from jax import lax
from jax.experimental import pallas as pl
from jax.experimental.pallas import tpu as pltpu

B, L, D_IN, D_H, D_OUT, TOPK = 4, 8192, 128, 64, 22, 8
D_PAD = 24
NCHUNK = 4
WINDOW = 3
CT = (B * L) // NCHUNK
CPB = L // CT
SUB = 1024


def _rowmax(g):
    g = jnp.maximum(g, pltpu.roll(g, 4, axis=0))
    g = jnp.maximum(g, pltpu.roll(g, 2, axis=0))
    return jnp.maximum(g, pltpu.roll(g, 1, axis=0))


def _mlp_topk_kernel(x_hbm, w1_ref, b1_ref, w2_ref, b2_ref, o_hbm,
                     x_buf, y_buf, w2p_ref, b2p_ref, in_sems, out_sems):
    w1 = w1_ref[...]
    b1 = b1_ref[...].reshape(1, D_H)
    w2p_ref[...] = jnp.zeros((D_PAD, D_H), jnp.float32)
    w2p_ref[0:D_OUT, :] = w2_ref[...]
    b2p_ref[...] = jnp.zeros((D_PAD, 1), jnp.float32)
    b2p_ref[0:D_OUT, :] = jnp.transpose(b2_ref[...].reshape(1, D_OUT))
    w2 = w2p_ref[...]
    b2 = b2p_ref[...]

    def start_in(i):
        b, j = divmod(i, CPB)
        cp = pltpu.make_async_copy(
            x_hbm.at[b, pl.ds(j * CT, CT), :], x_buf.at[i], in_sems.at[i])
        cp.start()
        return cp

    in_cps = [start_in(i) for i in range(WINDOW)]

    def process(i):
        for s in range(CT // SUB):
            tok = pl.ds(s * SUB, SUB)
            x = x_buf[i, tok, :]
            h = lax.dot_general(x, w1, (((1,), (1,)), ((), ())),
                                preferred_element_type=jnp.float32)
            h = jnp.maximum(h + b1, 0.0)
            y = lax.dot_general(w2, h, (((1,), (1,)), ((), ())),
                                preferred_element_type=jnp.float32)
            y = jnp.maximum(y + b2, 0.0)
            grp = (y[0:8], y[8:16], y[16:24])
            work = grp
            for _ in range(TOPK - 1):
                m = _rowmax(jnp.maximum(jnp.maximum(work[0], work[1]),
                                        work[2]))
                work = tuple(jnp.where(w >= m, -1.0, w) for w in work)
            t8 = _rowmax(jnp.maximum(jnp.maximum(work[0], work[1]), work[2]))
            y_buf[i, 0:8, tok] = jnp.where(grp[0] >= t8, grp[0], 0.0)
            y_buf[i, 8:16, tok] = jnp.where(grp[1] >= t8, grp[1], 0.0)
            y_buf[i, 16:22, tok] = jnp.where(grp[2] >= t8, grp[2], 0.0)[0:6]
        b, j = divmod(i, CPB)
        cp = pltpu.make_async_copy(
            y_buf.at[i],
            o_hbm.at[b, :, pl.ds(j * CT, CT)], out_sems.at[i])
        cp.start()
        return cp

    out_cps = []
    for i in range(NCHUNK):
        in_cps[i].wait()
        if i + WINDOW < NCHUNK:
            in_cps.append(start_in(i + WINDOW))
        out_cps.append(process(i))

    for cp in out_cps:
        cp.wait()


@jax.jit
def kernel(input, W1, b1, W2, b2):
    return pl.pallas_call(
        _mlp_topk_kernel,
        in_specs=[
            pl.BlockSpec(memory_space=pl.ANY),
            pl.BlockSpec(memory_space=pltpu.MemorySpace.VMEM),
            pl.BlockSpec(memory_space=pltpu.MemorySpace.VMEM),
            pl.BlockSpec(memory_space=pltpu.MemorySpace.VMEM),
            pl.BlockSpec(memory_space=pltpu.MemorySpace.VMEM),
        ],
        out_specs=pl.BlockSpec(memory_space=pl.ANY),
        out_shape=jax.ShapeDtypeStruct((B, D_OUT, L), jnp.float32),
        scratch_shapes=[
            pltpu.VMEM((NCHUNK, CT, D_IN), jnp.float32),
            pltpu.VMEM((NCHUNK, D_OUT, CT), jnp.float32),
            pltpu.VMEM((D_PAD, D_H), jnp.float32),
            pltpu.VMEM((D_PAD, 1), jnp.float32),
            pltpu.SemaphoreType.DMA((NCHUNK,)),
            pltpu.SemaphoreType.DMA((NCHUNK,)),
        ],
    )(input, W1, b1, W2, b2)
